# Initial kernel scaffold; baseline (speedup 1.0000x reference)
import jax
import jax.numpy as jnp
from jax.experimental import pallas as pl


def kernel(x, edge_index, W_gcn, b_gcn, W_ih, W_hh, b_ih, b_hh, W_lin, b_lin):
    raise NotImplementedError("write your pallas kernel here")


# speedup vs baseline: 2.9653x; 2.9653x over previous
import functools

import jax
import jax.numpy as jnp
from jax import lax
from jax.experimental import pallas as pl
from jax.experimental.pallas import tpu as pltpu

N_NODES = 15
N_EDGES = 256
NP = 16
HID = 512


def _fused_body(ei_ref, eiT_ref, xT_ref, WgT_ref, bg_ref,
                WihT_r_ref, WihT_z_ref, WihT_n_ref,
                WhhT_r_ref, WhhT_z_ref, WhhT_n_ref,
                B_r_ref, B_z_ref, Bih_n_ref, Bhh_n_ref,
                wlin_ref, blin_ref, out_ref,
                gi_r_ref, gi_z_ref, gi_n_ref):
    f32 = jnp.float32
    src_row = ei_ref[0:1, :]
    dst_row = ei_ref[1:2, :]
    src_col = eiT_ref[:, 0:1]
    dst_col = eiT_ref[:, 1:2]
    node_col = lax.broadcasted_iota(jnp.int32, (NP, 1), 0)
    node_row = lax.broadcasted_iota(jnp.int32, (1, NP), 1)
    ST = (src_row == node_col).astype(f32)
    DT = (dst_row == node_col).astype(f32)
    D = (dst_col == node_row).astype(f32)

    deg_col = jnp.sum(DT, axis=1, keepdims=True) + 1.0
    dinv_col = lax.rsqrt(deg_col)
    dinv_src = jnp.sum(ST * dinv_col, axis=0, keepdims=True)
    dinv_dst = jnp.sum(DT * dinv_col, axis=0, keepdims=True)
    norm_row = dinv_src * dinv_dst

    AT = jnp.dot(ST * norm_row, D, preferred_element_type=f32)
    eye = (node_col == node_row).astype(f32)
    AT = AT + eye * (1.0 / deg_col)

    aggT = jnp.dot(xT_ref[:, :], AT, preferred_element_type=f32)
    hT = jnp.dot(WgT_ref[:, :], aggT, preferred_element_type=f32)
    hT = jnp.maximum(hT + bg_ref[:, :], 0.0)

    gi_r_ref[:, :] = jnp.dot(hT, WihT_r_ref[:, :], preferred_element_type=f32) + B_r_ref[:, :]
    gi_z_ref[:, :] = jnp.dot(hT, WihT_z_ref[:, :], preferred_element_type=f32) + B_z_ref[:, :]
    gi_n_ref[:, :] = jnp.dot(hT, WihT_n_ref[:, :], preferred_element_type=f32) + Bih_n_ref[:, :]

    Whh_r = WhhT_r_ref[:, :]
    Whh_z = WhhT_z_ref[:, :]
    Whh_n = WhhT_n_ref[:, :]
    bhh_n = Bhh_n_ref[:, :]

    def step(t, carry):
        h, acc = carry
        gh_r = jnp.dot(h, Whh_r, preferred_element_type=f32)
        gh_z = jnp.dot(h, Whh_z, preferred_element_type=f32)
        gh_n = jnp.dot(h, Whh_n, preferred_element_type=f32) + bhh_n
        r = jax.nn.sigmoid(gi_r_ref[pl.ds(t, 1), :] + gh_r)
        z = jax.nn.sigmoid(gi_z_ref[pl.ds(t, 1), :] + gh_z)
        ng = jnp.tanh(gi_n_ref[pl.ds(t, 1), :] + r * gh_n)
        h = (1.0 - z) * ng + z * h
        acc = acc + h * wlin_ref[pl.ds(t, 1), 0:1]
        return h, acc

    h0 = jnp.zeros((1, NP), dtype=f32)
    acc0 = jnp.zeros((1, NP), dtype=f32)
    _, acc = lax.fori_loop(0, HID, step, (h0, acc0), unroll=4)
    out_ref[:, :] = acc + blin_ref[0:1, 0:1]


def _pad2(a, r, c):
    return jnp.pad(a, ((0, r - a.shape[0]), (0, c - a.shape[1])))


@functools.partial(jax.jit, static_argnames=())
def kernel(x, edge_index, W_gcn, b_gcn, W_ih, W_hh, b_ih, b_hh, W_lin, b_lin):
    f32 = jnp.float32
    Hd = W_hh.shape[1]
    ei = edge_index.astype(jnp.int32)
    eiT = ei.T
    xT = _pad2(x.T.astype(f32), HID, NP)
    WgT = W_gcn.T.astype(f32)
    bg = b_gcn.reshape(HID, 1).astype(f32)

    def gate(W, i):
        return _pad2(W[i * Hd:(i + 1) * Hd, :].T.astype(f32), NP, NP)

    WihT_r, WihT_z, WihT_n = gate(W_ih, 0), gate(W_ih, 1), gate(W_ih, 2)
    WhhT_r, WhhT_z, WhhT_n = gate(W_hh, 0), gate(W_hh, 1), gate(W_hh, 2)

    def brow(b):
        return _pad2(b.reshape(1, Hd).astype(f32), 1, NP)

    B_r = brow(b_ih[0:Hd] + b_hh[0:Hd])
    B_z = brow(b_ih[Hd:2 * Hd] + b_hh[Hd:2 * Hd])
    Bih_n = brow(b_ih[2 * Hd:])
    Bhh_n = brow(b_hh[2 * Hd:])
    wlin = W_lin.reshape(HID, 1).astype(f32)
    blin = b_lin.reshape(1, 1).astype(f32)

    acc = pl.pallas_call(
        _fused_body,
        out_shape=jax.ShapeDtypeStruct((1, NP), f32),
        scratch_shapes=[pltpu.VMEM((HID, NP), f32)] * 3,
    )(ei, eiT, xT, WgT, bg,
      WihT_r, WihT_z, WihT_n, WhhT_r, WhhT_z, WhhT_n,
      B_r, B_z, Bih_n, Bhh_n, wlin, blin)

    return acc[0, :N_NODES].reshape(N_NODES, 1)
